# baseline (speedup 1.0000x reference)
import functools

import jax
import jax.numpy as jnp
from jax.experimental import pallas as pl
from jax.experimental.pallas import tpu as pltpu

_IN_DIM = 768
_E_DIM = 64
_NUM_EMB = 256
_BETA = 0.001
_BM = 512


def _dot(a, b):
    return jax.lax.dot_general(a, b, (((1,), (0,)), ((), ())),
                               preferred_element_type=jnp.float32)


def _dot_t(a, b):
    return jax.lax.dot_general(a, b, (((1,), (1,)), ((), ())),
                               preferred_element_type=jnp.float32)


def _bf(x):
    return x.astype(jnp.bfloat16)


def _mlp(h, layers):
    n = len(layers)
    for li, (W, b) in enumerate(layers):
        h = _dot(h, W) + b
        if li < n - 1:
            h = _bf(jnp.maximum(h, 0.0))
    return h


def _prep_body(eW0, eW1, eW2, eW3, dW0, dW1, dW2, dW3,
               cb0, cb1, cb2, cb3,
               oW0, oW1, oW2, oW3, oD0, oD1, oD2, oD3,
               op0, op1, op2, op3, os0, os1, os2, os3):
    for src, dst in ((eW0, oW0), (eW1, oW1), (eW2, oW2), (eW3, oW3),
                     (dW0, oD0), (dW1, oD1), (dW2, oD2), (dW3, oD3)):
        dst[...] = _bf(src[...])
    for src, pdst, sdst in ((cb0, op0, os0), (cb1, op1, os1),
                            (cb2, op2, os2), (cb3, op3, os3)):
        cb = src[...]
        hi = _bf(cb)
        r1 = cb - hi.astype(jnp.float32)
        lo = _bf(r1)
        lolo = _bf(r1 - lo.astype(jnp.float32))
        pdst[...] = jnp.concatenate([hi, lo, lolo], axis=0)
        sdst[...] = jnp.sum(cb * cb, axis=1)[None, :]


def _prep(enc_Ws, dec_Ws, cbs):
    def whole(a):
        return pl.BlockSpec(a.shape, lambda: (0,) * a.ndim)

    operands = list(enc_Ws) + list(dec_Ws) + list(cbs)
    out_shapes = ([jax.ShapeDtypeStruct(W.shape, jnp.bfloat16)
                   for W in list(enc_Ws) + list(dec_Ws)]
                  + [jax.ShapeDtypeStruct((3 * _NUM_EMB, _E_DIM), jnp.bfloat16)
                     for _ in cbs]
                  + [jax.ShapeDtypeStruct((1, _NUM_EMB), jnp.float32)
                     for _ in cbs])
    out_specs = [whole(s) for s in out_shapes]
    return pl.pallas_call(
        _prep_body,
        in_specs=[whole(a) for a in operands],
        out_specs=out_specs,
        out_shape=out_shapes,
    )(*operands)


def _fused_body(n_blocks,
                x_ref, q_ref, w_ref,
                eW0, eb0, eW1, eb1, eW2, eb2, eW3, eb3,
                dW0, db0, dW1, db1, dW2, db2, dW3, db3,
                cp0, cs0, cp1, cs1, cp2, cs2, cp3, cs3,
                out_ref, xq_ref, idx_ref, rq_ref, qd_ref,
                xe_scr, rq_acc, qd_acc):
    i = pl.program_id(0)
    slot = jax.lax.rem(i, 2)

    enc = ((eW0[...], eb0[...]), (eW1[...], eb1[...]),
           (eW2[...], eb2[...]), (eW3[...], eb3[...]))
    dec = ((dW0[...], db0[...]), (dW1[...], db1[...]),
           (dW2[...], db2[...]), (dW3[...], db3[...]))
    planes = (cp0[...], cp1[...], cp2[...], cp3[...])
    sqs = (cs0[...], cs1[...], cs2[...], cs3[...])

    residual = xe_scr[pl.ds((1 - slot) * _BM, _BM), :]
    xq = jnp.zeros_like(residual)
    rq_sum = jnp.float32(0.0)
    idx_cols = []
    for cb3p, csq in zip(planes, sqs):
        cb_hi = cb3p[:_NUM_EMB, :]
        r_sq = jnp.sum(residual * residual, axis=1, keepdims=True)
        scores = (r_sq - 2.0 * _dot_t(_bf(residual), cb_hi)) + csq
        m = jnp.min(scores, axis=1, keepdims=True)
        lane = jax.lax.broadcasted_iota(jnp.int32, scores.shape, 1)
        idx2d = jnp.min(jnp.where(scores == m, lane, _NUM_EMB),
                        axis=1, keepdims=True)
        one_hot = (lane == idx2d).astype(jnp.bfloat16)
        oh3 = jnp.concatenate([one_hot, one_hot, one_hot], axis=1)
        qv = _dot(oh3, cb3p)
        diff = qv - residual
        rq_sum = rq_sum + jnp.sum(diff * diff)
        residual = -diff
        xq = xq + qv
        idx_cols.append(idx2d)

    out_ref[...] = _mlp(_bf(xq), dec)
    xq_ref[...] = xq
    idx_ref[...] = jnp.concatenate(idx_cols, axis=1)
    acc_rq = (jnp.where(i == 0, 0.0, rq_acc[0])
              + jnp.where(i > 0, rq_sum, 0.0))
    rq_acc[0] = acc_rq
    b_total = jnp.float32(n_blocks * _BM)
    rq_ref[...] = (acc_rq * ((1.0 + _BETA) / (4.0 * b_total * _E_DIM))
                   ).reshape(1, 1)

    h = _bf(jnp.concatenate([x_ref[...], q_ref[...]], axis=0))
    h = _mlp(h, enc)
    x_e = h[:_BM, :]
    q_enc = h[_BM:, :]
    xe_scr[pl.ds(slot * _BM, _BM), :] = x_e

    n1sq = jnp.sum(x_e * x_e, axis=1, keepdims=True)
    n2sq = jnp.sum(q_enc * q_enc, axis=1, keepdims=True)
    dotp = jnp.sum(x_e * q_enc, axis=1, keepdims=True)
    cos = dotp / jnp.maximum(jnp.sqrt(n1sq) * jnp.sqrt(n2sq), 1e-8)
    qd_sum = jnp.sum(w_ref[...] * cos)
    acc_qd = (jnp.where(i == 0, 0.0, qd_acc[0])
              + jnp.where(i < n_blocks, qd_sum, 0.0))
    qd_acc[0] = acc_qd
    qd_ref[...] = (1.0 - acc_qd / b_total).reshape(1, 1)


def kernel(x, q_embs, labels, qd_align_w,
           enc_W0, enc_b0, enc_W1, enc_b1, enc_W2, enc_b2, enc_W3, enc_b3,
           dec_W0, dec_b0, dec_W1, dec_b1, dec_W2, dec_b2, dec_W3, dec_b3,
           cb0, cb1, cb2, cb3):
    B = x.shape[0]
    n_blocks = B // _BM
    n_steps = n_blocks + 1
    enc_Ws = (enc_W0, enc_W1, enc_W2, enc_W3)
    enc_bs = (enc_b0, enc_b1, enc_b2, enc_b3)
    dec_Ws = (dec_W0, dec_W1, dec_W2, dec_W3)
    dec_bs = (dec_b0, dec_b1, dec_b2, dec_b3)

    prep = _prep(enc_Ws, dec_Ws, (cb0, cb1, cb2, cb3))
    enc_Wb, dec_Wb = prep[0:4], prep[4:8]
    cb_planes, cb_sqs = prep[8:12], prep[12:16]

    def in_spec(d):
        return pl.BlockSpec((_BM, d),
                            lambda i: (jnp.minimum(i, n_blocks - 1), 0))

    def skew_spec(d):
        return pl.BlockSpec((_BM, d),
                            lambda i: (jnp.maximum(i - 1, 0), 0))

    def whole(a):
        return pl.BlockSpec(a.shape, lambda i: (0,) * a.ndim)

    in_specs = [in_spec(_IN_DIM), in_spec(_IN_DIM), in_spec(1)]
    operands = [x, q_embs, qd_align_w.reshape(B, 1)]
    for W, b in zip(enc_Wb + dec_Wb, enc_bs + dec_bs):
        operands += [W, b.reshape(1, -1)]
        in_specs += [whole(W), pl.BlockSpec((1, b.shape[0]), lambda i: (0, 0))]
    for planes, csq in zip(cb_planes, cb_sqs):
        operands += [planes, csq]
        in_specs += [whole(planes),
                     pl.BlockSpec((1, _NUM_EMB), lambda i: (0, 0))]

    scalar_spec = pl.BlockSpec((1, 1), lambda i: (0, 0))
    out_shapes = (
        jax.ShapeDtypeStruct((B, _IN_DIM), jnp.float32),
        jax.ShapeDtypeStruct((B, _E_DIM), jnp.float32),
        jax.ShapeDtypeStruct((B, 4), jnp.int32),
        jax.ShapeDtypeStruct((1, 1), jnp.float32),
        jax.ShapeDtypeStruct((1, 1), jnp.float32),
    )
    out_specs = (
        skew_spec(_IN_DIM),
        skew_spec(_E_DIM),
        skew_spec(4),
        scalar_spec,
        scalar_spec,
    )

    out, x_q, indices, rq, qd = pl.pallas_call(
        functools.partial(_fused_body, n_blocks),
        grid=(n_steps,),
        in_specs=in_specs,
        out_specs=out_specs,
        out_shape=out_shapes,
        scratch_shapes=[pltpu.VMEM((2 * _BM, _E_DIM), jnp.float32),
                        pltpu.SMEM((1,), jnp.float32),
                        pltpu.SMEM((1,), jnp.float32)],
        compiler_params=pltpu.CompilerParams(
            vmem_limit_bytes=128 * 1024 * 1024,
        ),
    )(*operands)

    zeros4 = jnp.zeros((4,), jnp.float32)
    return (out, rq[0, 0], indices, x_q, zeros4, zeros4, qd[0, 0])


# speedup vs baseline: 1.1613x; 1.1613x over previous
import functools

import jax
import jax.numpy as jnp
from jax.experimental import pallas as pl
from jax.experimental.pallas import tpu as pltpu

_IN_DIM = 768
_E_DIM = 64
_NUM_EMB = 256
_BETA = 0.001
_BM = 1024


def _dot(a, b):
    return jax.lax.dot_general(a, b, (((1,), (0,)), ((), ())),
                               preferred_element_type=jnp.float32)


def _dot_t(a, b):
    return jax.lax.dot_general(a, b, (((1,), (1,)), ((), ())),
                               preferred_element_type=jnp.float32)


def _bf(x):
    return x.astype(jnp.bfloat16)


def _mlp(h, layers):
    n = len(layers)
    for li, (W, b) in enumerate(layers):
        h = _dot(h, W) + b
        if li < n - 1:
            h = _bf(jnp.maximum(h, 0.0))
    return h


def _prep_body(eW0, eW1, eW2, eW3, dW0, dW1, dW2, dW3,
               cb0, cb1, cb2, cb3,
               oW0, oW1, oW2, oW3, oD0, oD1, oD2, oD3,
               op0, op1, op2, op3, os0, os1, os2, os3):
    for src, dst in ((eW0, oW0), (eW1, oW1), (eW2, oW2), (eW3, oW3),
                     (dW0, oD0), (dW1, oD1), (dW2, oD2), (dW3, oD3)):
        dst[...] = _bf(src[...])
    for src, pdst, sdst in ((cb0, op0, os0), (cb1, op1, os1),
                            (cb2, op2, os2), (cb3, op3, os3)):
        cb = src[...]
        hi = _bf(cb)
        r1 = cb - hi.astype(jnp.float32)
        lo = _bf(r1)
        lolo = _bf(r1 - lo.astype(jnp.float32))
        pdst[...] = jnp.concatenate([hi, lo, lolo], axis=0)
        sdst[...] = jnp.sum(cb * cb, axis=1)[None, :]


def _prep(enc_Ws, dec_Ws, cbs):
    def whole(a):
        return pl.BlockSpec(a.shape, lambda: (0,) * a.ndim)

    operands = list(enc_Ws) + list(dec_Ws) + list(cbs)
    out_shapes = ([jax.ShapeDtypeStruct(W.shape, jnp.bfloat16)
                   for W in list(enc_Ws) + list(dec_Ws)]
                  + [jax.ShapeDtypeStruct((3 * _NUM_EMB, _E_DIM), jnp.bfloat16)
                     for _ in cbs]
                  + [jax.ShapeDtypeStruct((1, _NUM_EMB), jnp.float32)
                     for _ in cbs])
    out_specs = [whole(s) for s in out_shapes]
    return pl.pallas_call(
        _prep_body,
        in_specs=[whole(a) for a in operands],
        out_specs=out_specs,
        out_shape=out_shapes,
    )(*operands)


def _fused_body(n_blocks,
                x_ref, q_ref, w_ref,
                eW0, eb0, eW1, eb1, eW2, eb2, eW3, eb3,
                dW0, db0, dW1, db1, dW2, db2, dW3, db3,
                cb0_ref, cb1_ref, cb2_ref, cb3_ref,
                out_ref, xq_ref, idx_ref, rq_ref, qd_ref,
                xe_scr, rq_acc, qd_acc,
                we0, we1, we2, we3, wd0, wd1, wd2, wd3,
                pl0, pl1, pl2, pl3, sq0, sq1, sq2, sq3):
    i = pl.program_id(0)
    slot = jax.lax.rem(i, 2)

    @pl.when(i == 0)
    def _convert():
        for src, dst in ((eW0, we0), (eW1, we1), (eW2, we2), (eW3, we3),
                         (dW0, wd0), (dW1, wd1), (dW2, wd2), (dW3, wd3)):
            dst[...] = _bf(src[...])
        for src, pdst, sdst in ((cb0_ref, pl0, sq0), (cb1_ref, pl1, sq1),
                                (cb2_ref, pl2, sq2), (cb3_ref, pl3, sq3)):
            cb = src[...]
            hi = _bf(cb)
            r1 = cb - hi.astype(jnp.float32)
            lo = _bf(r1)
            lolo = _bf(r1 - lo.astype(jnp.float32))
            pdst[...] = jnp.concatenate([hi, lo, lolo], axis=0)
            sdst[...] = jnp.sum(cb * cb, axis=1)[None, :]

    enc = ((we0[...], eb0[...]), (we1[...], eb1[...]),
           (we2[...], eb2[...]), (we3[...], eb3[...]))
    dec = ((wd0[...], db0[...]), (wd1[...], db1[...]),
           (wd2[...], db2[...]), (wd3[...], db3[...]))
    planes = (pl0[...], pl1[...], pl2[...], pl3[...])
    sqs = (sq0[...], sq1[...], sq2[...], sq3[...])

    residual = xe_scr[pl.ds((1 - slot) * _BM, _BM), :]
    xq = jnp.zeros_like(residual)
    rq_sum = jnp.float32(0.0)
    idx_cols = []
    for cb3p, csq in zip(planes, sqs):
        cb_hi = cb3p[:_NUM_EMB, :]
        r_sq = jnp.sum(residual * residual, axis=1, keepdims=True)
        scores = (r_sq - 2.0 * _dot_t(_bf(residual), cb_hi)) + csq
        m = jnp.min(scores, axis=1, keepdims=True)
        lane = jax.lax.broadcasted_iota(jnp.int32, scores.shape, 1)
        idx2d = jnp.min(jnp.where(scores == m, lane, _NUM_EMB),
                        axis=1, keepdims=True)
        one_hot = (lane == idx2d).astype(jnp.bfloat16)
        oh3 = jnp.concatenate([one_hot, one_hot, one_hot], axis=1)
        qv = _dot(oh3, cb3p)
        diff = qv - residual
        rq_sum = rq_sum + jnp.sum(diff * diff)
        residual = -diff
        xq = xq + qv
        idx_cols.append(idx2d)

    out_ref[...] = _mlp(_bf(xq), dec)
    xq_ref[...] = xq
    idx_ref[...] = jnp.concatenate(idx_cols, axis=1)
    acc_rq = (jnp.where(i == 0, 0.0, rq_acc[0])
              + jnp.where(i > 0, rq_sum, 0.0))
    rq_acc[0] = acc_rq
    b_total = jnp.float32(n_blocks * _BM)
    rq_ref[...] = (acc_rq * ((1.0 + _BETA) / (4.0 * b_total * _E_DIM))
                   ).reshape(1, 1)

    h = _bf(jnp.concatenate([x_ref[...], q_ref[...]], axis=0))
    h = _mlp(h, enc)
    x_e = h[:_BM, :]
    q_enc = h[_BM:, :]
    xe_scr[pl.ds(slot * _BM, _BM), :] = x_e

    n1sq = jnp.sum(x_e * x_e, axis=1, keepdims=True)
    n2sq = jnp.sum(q_enc * q_enc, axis=1, keepdims=True)
    dotp = jnp.sum(x_e * q_enc, axis=1, keepdims=True)
    cos = dotp / jnp.maximum(jnp.sqrt(n1sq) * jnp.sqrt(n2sq), 1e-8)
    qd_sum = jnp.sum(w_ref[...] * cos)
    acc_qd = (jnp.where(i == 0, 0.0, qd_acc[0])
              + jnp.where(i < n_blocks, qd_sum, 0.0))
    qd_acc[0] = acc_qd
    qd_ref[...] = (1.0 - acc_qd / b_total).reshape(1, 1)


def kernel(x, q_embs, labels, qd_align_w,
           enc_W0, enc_b0, enc_W1, enc_b1, enc_W2, enc_b2, enc_W3, enc_b3,
           dec_W0, dec_b0, dec_W1, dec_b1, dec_W2, dec_b2, dec_W3, dec_b3,
           cb0, cb1, cb2, cb3):
    B = x.shape[0]
    n_blocks = B // _BM
    n_steps = n_blocks + 1
    enc_Ws = (enc_W0, enc_W1, enc_W2, enc_W3)
    enc_bs = (enc_b0, enc_b1, enc_b2, enc_b3)
    dec_Ws = (dec_W0, dec_W1, dec_W2, dec_W3)
    dec_bs = (dec_b0, dec_b1, dec_b2, dec_b3)

    def in_spec(d):
        return pl.BlockSpec((_BM, d),
                            lambda i: (jnp.minimum(i, n_blocks - 1), 0))

    def skew_spec(d):
        return pl.BlockSpec((_BM, d),
                            lambda i: (jnp.maximum(i - 1, 0), 0))

    def whole(a):
        return pl.BlockSpec(a.shape, lambda i: (0,) * a.ndim)

    in_specs = [in_spec(_IN_DIM), in_spec(_IN_DIM), in_spec(1)]
    operands = [x, q_embs, qd_align_w.reshape(B, 1)]
    for W, b in zip(enc_Ws + dec_Ws, enc_bs + dec_bs):
        operands += [W, b.reshape(1, -1)]
        in_specs += [whole(W), pl.BlockSpec((1, b.shape[0]), lambda i: (0, 0))]
    for cb in (cb0, cb1, cb2, cb3):
        operands.append(cb)
        in_specs.append(whole(cb))

    scalar_spec = pl.BlockSpec((1, 1), lambda i: (0, 0))
    out_shapes = (
        jax.ShapeDtypeStruct((B, _IN_DIM), jnp.float32),
        jax.ShapeDtypeStruct((B, _E_DIM), jnp.float32),
        jax.ShapeDtypeStruct((B, 4), jnp.int32),
        jax.ShapeDtypeStruct((1, 1), jnp.float32),
        jax.ShapeDtypeStruct((1, 1), jnp.float32),
    )
    out_specs = (
        skew_spec(_IN_DIM),
        skew_spec(_E_DIM),
        skew_spec(4),
        scalar_spec,
        scalar_spec,
    )

    out, x_q, indices, rq, qd = pl.pallas_call(
        functools.partial(_fused_body, n_blocks),
        grid=(n_steps,),
        in_specs=in_specs,
        out_specs=out_specs,
        out_shape=out_shapes,
        scratch_shapes=(
            [pltpu.VMEM((2 * _BM, _E_DIM), jnp.float32),
             pltpu.SMEM((1,), jnp.float32),
             pltpu.SMEM((1,), jnp.float32)]
            + [pltpu.VMEM(W.shape, jnp.bfloat16) for W in enc_Ws + dec_Ws]
            + [pltpu.VMEM((3 * _NUM_EMB, _E_DIM), jnp.bfloat16)
               for _ in range(4)]
            + [pltpu.VMEM((1, _NUM_EMB), jnp.float32) for _ in range(4)]),
        compiler_params=pltpu.CompilerParams(
            vmem_limit_bytes=128 * 1024 * 1024,
        ),
    )(*operands)

    zeros4 = jnp.zeros((4,), jnp.float32)
    return (out, rq[0, 0], indices, x_q, zeros4, zeros4, qd[0, 0])
